# baseline (speedup 1.0000x reference)
import functools

import jax
import jax.numpy as jnp
from jax.experimental import pallas as pl
from jax.experimental.pallas import tpu as pltpu

B = 16
C = 128
LF = 4096
ANCHOR_LENGTHS = (2.0, 4.0, 6.0, 9.0, 13.0, 18.0)
A = len(ANCHOR_LENGTHS)
RQ = LF // 128
BPB = 2
TS = 2
LT = LF // TS
RQT = LT // 128


def _interleave(rows, p_ref, n):
    r3 = rows.astype(jnp.bfloat16).reshape(n, rows.shape[1] // 128, 128)
    acc = jnp.dot(r3[0], p_ref[0], preferred_element_type=jnp.float32)
    for ch in range(1, n):
        acc += jnp.dot(r3[ch], p_ref[ch], preferred_element_type=jnp.float32)
    return acc


def _rpn_kernel(feat_ref, bnd_ref, w2_ref, cb_ref, ow_ref, ob_ref, rw_ref,
                rb_ref, aoff_ref, pobj_ref, obj_ref, reg_ref, anch_ref):
    t = pl.program_id(1)
    for bb in range(BPB):
        x = feat_ref[bb].astype(jnp.bfloat16)
        bnd = bnd_ref[bb].astype(jnp.bfloat16)
        zero = jnp.zeros((C, 1), jnp.bfloat16)
        first = jnp.where(t == 0, zero, bnd[:, 0:1])
        last = jnp.where(t == 0, bnd[:, 1:2], zero)
        xr = jnp.concatenate([first, x[:, :-1]], axis=1)
        xl = jnp.concatenate([x[:, 1:], last], axis=1)
        x3 = jnp.concatenate([xr, x, xl], axis=0)

        h = jnp.dot(w2_ref[:].astype(jnp.bfloat16), x3,
                    preferred_element_type=jnp.float32)
        h = jnp.maximum(h + cb_ref[:], 0.0)
        hb = h.astype(jnp.bfloat16)

        obj = jnp.dot(ow_ref[:].astype(jnp.bfloat16), hb,
                      preferred_element_type=jnp.float32)
        reg = jnp.dot(rw_ref[:].astype(jnp.bfloat16), hb,
                      preferred_element_type=jnp.float32)
        obj_ref[bb] = _interleave(obj + ob_ref[:], pobj_ref, A)
        regb = reg + rb_ref[:]
        reg_ref[bb, 0] = _interleave(regb[:A], pobj_ref, A)
        reg_ref[bb, 1] = _interleave(regb[A:], pobj_ref, A)

    @pl.when(pl.program_id(0) == 0)
    def _():
        shape = (RQT, A * 128)
        row = jax.lax.broadcasted_iota(jnp.int32, shape, 0)
        col = jax.lax.broadcasted_iota(jnp.int32, shape, 1)
        centers = (128 * (RQT * t + row) + col // A).astype(jnp.float32) + 0.5
        anch_ref[0] = centers + aoff_ref[0]
        anch_ref[1] = centers + aoff_ref[1]


def _perm(n):
    j = jnp.arange(128)
    q = jnp.arange(n * 128)
    ch = jnp.arange(n)
    return (q[None, None, :] == n * j[None, :, None] + ch[:, None, None]
            ).astype(jnp.bfloat16)


@functools.partial(jax.jit, static_argnames=())
def kernel(feat, conv_w, conv_b, obj_w, obj_b, reg_w, reg_b):
    w2 = jnp.transpose(conv_w, (0, 2, 1)).reshape(C, 3 * C)
    cb = conv_b.reshape(C, 1)
    ow = obj_w[:, :, 0]
    ob = obj_b.reshape(A, 1)
    dperm = jnp.concatenate([jnp.arange(0, 2 * A, 2), jnp.arange(1, 2 * A, 2)])
    rw = reg_w[:, :, 0][dperm]
    rb = reg_b[dperm].reshape(2 * A, 1)
    lens = jnp.repeat(jnp.asarray(ANCHOR_LENGTHS, jnp.float32), 2)
    sign = jnp.tile(jnp.asarray([-0.5, 0.5], jnp.float32), A)
    arow = sign * lens
    q6 = jnp.arange(A * 128) % A
    aoff = arow[2 * q6[None, :] + jnp.arange(2)[:, None]].reshape(2, 1, A * 128)
    pobj = _perm(A)

    obj, reg, anch = pl.pallas_call(
        _rpn_kernel,
        grid=(B // BPB, TS),
        in_specs=[
            pl.BlockSpec((BPB, C, LT), lambda b, t: (b, 0, t)),
            pl.BlockSpec((BPB, C, 2), lambda b, t: (b, 0, 0)),
            pl.BlockSpec((C, 3 * C), lambda b, t: (0, 0)),
            pl.BlockSpec((C, 1), lambda b, t: (0, 0)),
            pl.BlockSpec((A, C), lambda b, t: (0, 0)),
            pl.BlockSpec((A, 1), lambda b, t: (0, 0)),
            pl.BlockSpec((2 * A, C), lambda b, t: (0, 0)),
            pl.BlockSpec((2 * A, 1), lambda b, t: (0, 0)),
            pl.BlockSpec((2, 1, A * 128), lambda b, t: (0, 0, 0)),
            pl.BlockSpec((A, 128, A * 128), lambda b, t: (0, 0, 0)),
        ],
        out_specs=[
            pl.BlockSpec((BPB, RQT, A * 128), lambda b, t: (b, t, 0)),
            pl.BlockSpec((BPB, 2, RQT, A * 128), lambda b, t: (b, 0, t, 0)),
            pl.BlockSpec((2, RQT, A * 128), lambda b, t: (0, t, 0)),
        ],
        out_shape=[
            jax.ShapeDtypeStruct((B, RQ, A * 128), jnp.float32),
            jax.ShapeDtypeStruct((B, 2, RQ, A * 128), jnp.float32),
            jax.ShapeDtypeStruct((2, RQ, A * 128), jnp.float32),
        ],
    )(feat, jax.lax.slice_in_dim(feat, LT - 1, LT + 1, axis=2),
      w2, cb, ow, ob, rw, rb, aoff, pobj)

    return (obj.reshape(B, LF * A),
            jnp.transpose(reg.reshape(B, 2, LF * A), (0, 2, 1)),
            jnp.transpose(anch.reshape(2, LF * A), (1, 0)))


# speedup vs baseline: 1.3685x; 1.3685x over previous
import functools

import jax
import jax.numpy as jnp
from jax.experimental import pallas as pl
from jax.experimental.pallas import tpu as pltpu

B = 16
C = 128
LF = 4096
ANCHOR_LENGTHS = (2.0, 4.0, 6.0, 9.0, 13.0, 18.0)
A = len(ANCHOR_LENGTHS)
RQ = LF // 128
BPB = 2


def _interleave(rows, p_ref, n):
    r3 = rows.astype(jnp.bfloat16).reshape(n, RQ, 128)
    acc = jnp.dot(r3[0], p_ref[0], preferred_element_type=jnp.float32)
    for ch in range(1, n):
        acc += jnp.dot(r3[ch], p_ref[ch], preferred_element_type=jnp.float32)
    return acc


def _rpn_kernel(feat_ref, w2_ref, cb_ref, ow_ref, ob_ref, rw_ref,
                rb_ref, aoff_ref, pobj_ref, obj_ref, reg_ref, anch_ref):
    for bb in range(BPB):
        x = feat_ref[bb].astype(jnp.bfloat16)
        zero = jnp.zeros((C, 1), jnp.bfloat16)
        xr = jnp.concatenate([zero, x[:, :-1]], axis=1)
        xl = jnp.concatenate([x[:, 1:], zero], axis=1)
        x3 = jnp.concatenate([xr, x, xl], axis=0)

        h = jnp.dot(w2_ref[:].astype(jnp.bfloat16), x3,
                    preferred_element_type=jnp.float32)
        h = jnp.maximum(h + cb_ref[:], 0.0)
        hb = h.astype(jnp.bfloat16)

        obj = jnp.dot(ow_ref[:].astype(jnp.bfloat16), hb,
                      preferred_element_type=jnp.float32)
        reg = jnp.dot(rw_ref[:].astype(jnp.bfloat16), hb,
                      preferred_element_type=jnp.float32)
        tobj = _interleave(obj + ob_ref[:], pobj_ref, A)
        brow = 2 * (pl.program_id(0) % 4) + bb
        for i in range(RQ):
            obj_ref[0, pl.ds(brow, 1), pl.ds(768 * i, 768)] = tobj[i:i + 1, :]
        regb = reg + rb_ref[:]
        reg_ref[bb, 0] = _interleave(regb[:A], pobj_ref, A)
        reg_ref[bb, 1] = _interleave(regb[A:], pobj_ref, A)

    @pl.when(pl.program_id(0) == 0)
    def _():
        shape = (RQ, A * 128)
        row = jax.lax.broadcasted_iota(jnp.int32, shape, 0)
        col = jax.lax.broadcasted_iota(jnp.int32, shape, 1)
        centers = (128 * row + col // A).astype(jnp.float32) + 0.5
        anch_ref[0] = centers + aoff_ref[0]
        anch_ref[1] = centers + aoff_ref[1]


def _perm(n):
    j = jnp.arange(128)
    q = jnp.arange(n * 128)
    ch = jnp.arange(n)
    return (q[None, None, :] == n * j[None, :, None] + ch[:, None, None]
            ).astype(jnp.bfloat16)


@functools.partial(jax.jit, static_argnames=())
def kernel(feat, conv_w, conv_b, obj_w, obj_b, reg_w, reg_b):
    w2 = jnp.transpose(conv_w, (0, 2, 1)).reshape(C, 3 * C)
    cb = conv_b.reshape(C, 1)
    ow = obj_w[:, :, 0]
    ob = obj_b.reshape(A, 1)
    dperm = jnp.concatenate([jnp.arange(0, 2 * A, 2), jnp.arange(1, 2 * A, 2)])
    rw = reg_w[:, :, 0][dperm]
    rb = reg_b[dperm].reshape(2 * A, 1)
    lens = jnp.repeat(jnp.asarray(ANCHOR_LENGTHS, jnp.float32), 2)
    sign = jnp.tile(jnp.asarray([-0.5, 0.5], jnp.float32), A)
    arow = sign * lens
    q6 = jnp.arange(A * 128) % A
    aoff = arow[2 * q6[None, :] + jnp.arange(2)[:, None]].reshape(2, 1, A * 128)
    pobj = _perm(A)

    obj, reg, anch = pl.pallas_call(
        _rpn_kernel,
        grid=(B // BPB,),
        in_specs=[
            pl.BlockSpec((BPB, C, LF), lambda b: (b, 0, 0)),
            pl.BlockSpec((C, 3 * C), lambda b: (0, 0)),
            pl.BlockSpec((C, 1), lambda b: (0, 0)),
            pl.BlockSpec((A, C), lambda b: (0, 0)),
            pl.BlockSpec((A, 1), lambda b: (0, 0)),
            pl.BlockSpec((2 * A, C), lambda b: (0, 0)),
            pl.BlockSpec((2 * A, 1), lambda b: (0, 0)),
            pl.BlockSpec((2, 1, A * 128), lambda b: (0, 0, 0)),
            pl.BlockSpec((A, 128, A * 128), lambda b: (0, 0, 0)),
        ],
        out_specs=[
            pl.BlockSpec((1, 8, LF * A), lambda b: (b // 4, 0, 0)),
            pl.BlockSpec((BPB, 2, RQ, A * 128), lambda b: (b, 0, 0, 0)),
            pl.BlockSpec((2, RQ, A * 128), lambda b: (0, 0, 0)),
        ],
        out_shape=[
            jax.ShapeDtypeStruct((B // 8, 8, LF * A), jnp.float32),
            jax.ShapeDtypeStruct((B, 2, RQ, A * 128), jnp.float32),
            jax.ShapeDtypeStruct((2, RQ, A * 128), jnp.float32),
        ],
    )(feat, w2, cb, ow, ob, rw, rb, aoff, pobj)

    return (obj.reshape(B, LF * A),
            jnp.transpose(reg.reshape(B, 2, LF * A), (0, 2, 1)),
            jnp.transpose(anch.reshape(2, LF * A), (1, 0)))
